# baseline (speedup 1.0000x reference)
import functools

import jax
import jax.numpy as jnp
from jax import lax
from jax.experimental import pallas as pl
from jax.experimental.pallas import tpu as pltpu
from jax.experimental.pallas import tpu_sc as plsc

N_NODES = 10000
N_EDGES = 320000
D_FEAT = 128
D_EDGE = 16
HIDDEN = 128
OUT = 128

NC = 2
NS = 16
DH = D_FEAT // NC
CHUNK = 128
TOT_CHUNKS = N_EDGES // CHUNK
BASE_CH = TOT_CHUNKS // NS
EXTRA = TOT_CHUNKS - BASE_CH * NS
T_MAX = BASE_CH + 1
T_LOOP = 158
UNROLL = 2
EROWS = CHUNK * D_EDGE // 128
EAP_ROWS = N_EDGES * D_EDGE // 128
IDX_ROWS = 2504
EDGE_SPLIT = TOT_CHUNKS // NC
NZ_S = 10
PK_N = N_NODES * DH // 128
PK_E = N_NODES * D_EDGE // 128
ZROWS = 125


def _sc_agg_body(rxlo_hbm, rxhi_hbm, src_hbm, dst_hbm, ea_hbm,
                 outn_hbm, oute_hbm,
                 src_v, dst_v, rows, ebuf, zbuf, zbuf16,
                 accn_sh, acce_sh, gsem, ssem, esem, tsem):
    cid = lax.axis_index("c")
    sid = lax.axis_index("s")

    zero16 = jnp.zeros((16,), jnp.float32)

    def _zrow(i, carry):
        for k in range(DH // 16):
            zbuf[i, pl.ds(k * 16, 16)] = zero16
        zbuf16[i, :] = zero16
        return carry

    rows_z = N_NODES // NZ_S

    @pl.when(sid < NZ_S)
    def _zero_fill():
        lax.fori_loop(0, ZROWS, _zrow, 0)
        for j in range(rows_z // ZROWS):
            r0 = sid * rows_z + j * ZROWS
            pltpu.sync_copy(zbuf, accn_sh.at[pl.ds(r0, ZROWS)])
            pltpu.sync_copy(zbuf16, acce_sh.at[pl.ds(r0, ZROWS)])

    plsc.subcore_barrier()

    accn_nodes = accn_sh
    acce_nodes = acce_sh

    start = sid * BASE_CH + jnp.minimum(sid, EXTRA)
    n_t = BASE_CH + (sid < EXTRA).astype(jnp.int32)

    pltpu.sync_copy(src_hbm.at[pl.ds(start, T_MAX)], src_v)
    pltpu.sync_copy(dst_hbm.at[pl.ds(start, T_MAX)], dst_v)

    def _issue_gather(t, q):
        @pl.when(cid == 0)
        def _g_lo():
            pltpu.async_copy(rxlo_hbm.at[src_v.at[t]], rows.at[q], gsem[q])

        @pl.when(cid == 1)
        def _g_hi():
            pltpu.async_copy(rxhi_hbm.at[src_v.at[t]], rows.at[q], gsem[q])

    def _edge_active(t):
        return ((start + t) // EDGE_SPLIT) == cid

    def _issue_edge(t, q):
        @pl.when(_edge_active(t))
        def _e():
            pltpu.async_copy(ea_hbm.at[pl.ds((start + t) * CHUNK, CHUNK)],
                             ebuf.at[q], esem[q])

    _issue_gather(0, 0)
    _issue_edge(0, 0)

    def _step(s, carry):
        for q in range(UNROLL):
            t = UNROLL * s + q
            q1 = (q + 1) % UNROLL

            @pl.when((t >= 1) & (t - 1 < n_t))
            def _drain():
                pltpu.make_async_copy(
                    rows.at[q1], accn_nodes.at[dst_v.at[t - 1]], ssem[q1]
                ).wait()

                @pl.when(_edge_active(t - 1))
                def _drain_e():
                    pltpu.make_async_copy(
                        ebuf.at[q1], acce_nodes.at[dst_v.at[t - 1]], tsem[q1]
                    ).wait()

            @pl.when(t + 1 < n_t)
            def _issue_next():
                _issue_gather(t + 1, q1)
                _issue_edge(t + 1, q1)

            @pl.when(t < n_t)
            def _node_part():
                pltpu.make_async_copy(rxlo_hbm.at[src_v.at[t]], rows.at[q],
                                      gsem[q]).wait()
                pltpu.async_copy(rows.at[q], accn_nodes.at[dst_v.at[t]],
                                 ssem[q], add=True)

            @pl.when((t < n_t) & _edge_active(t))
            def _edge_part():
                pltpu.make_async_copy(
                    ea_hbm.at[pl.ds((start + t) * CHUNK, CHUNK)],
                    ebuf.at[q], esem[q]).wait()

                def _relu_row(r, carry):
                    for k in range(8):
                        i = r * 8 + k
                        ebuf[q, i, :] = jnp.maximum(ebuf[q, i, :], 0.0)
                    return carry

                lax.fori_loop(0, CHUNK // 8, _relu_row, 0)
                pltpu.async_copy(ebuf.at[q], acce_nodes.at[dst_v.at[t]],
                                 tsem[q], add=True)

        return carry

    lax.fori_loop(0, T_LOOP // UNROLL, _step, 0)
    plsc.subcore_barrier()

    @pl.when(sid < NZ_S)
    def _copy_out():
        r0 = sid * rows_z
        pltpu.sync_copy(accn_sh.at[pl.ds(r0, rows_z)],
                        outn_hbm.at[cid, pl.ds(r0, rows_z)])
        pltpu.sync_copy(acce_sh.at[pl.ds(r0, rows_z)],
                        oute_hbm.at[cid, pl.ds(r0, rows_z)])


_sc_agg = functools.partial(
    pl.kernel,
    out_type=(jax.ShapeDtypeStruct((NC, N_NODES, DH), jnp.float32),
              jax.ShapeDtypeStruct((NC, N_NODES, D_EDGE), jnp.float32)),
    mesh=plsc.VectorSubcoreMesh(core_axis_name="c", subcore_axis_name="s"),
    compiler_params=pltpu.CompilerParams(use_tc_tiling_on_sc=False, internal_scratch_in_bytes=131072),
    scratch_types=[
        pltpu.VMEM((T_MAX, CHUNK), jnp.int32),
        pltpu.VMEM((T_MAX, CHUNK), jnp.int32),
        pltpu.VMEM((UNROLL, CHUNK, DH), jnp.float32),
        pltpu.VMEM((UNROLL, CHUNK, D_EDGE), jnp.float32),
        pltpu.VMEM((ZROWS, DH), jnp.float32),
        pltpu.VMEM((ZROWS, D_EDGE), jnp.float32),
        pltpu.VMEM_SHARED((N_NODES, DH), jnp.float32),
        pltpu.VMEM_SHARED((N_NODES, D_EDGE), jnp.float32),
        [pltpu.SemaphoreType.DMA] * UNROLL,
        [pltpu.SemaphoreType.DMA] * UNROLL,
        [pltpu.SemaphoreType.DMA] * UNROLL,
        [pltpu.SemaphoreType.DMA] * UNROLL,
    ],
)(_sc_agg_body)


def _prep_body(x_ref, rxlo_ref, rxhi_ref):
    rx = jnp.maximum(x_ref[...], 0.0)
    rxlo_ref[...] = rx[:, :DH]
    rxhi_ref[...] = rx[:, DH:]


def _mlp_body(an, ae, xlo, xhi, w1lo, w1hi, w1e, b1, w2, b2, o_ref):
    hlo = an[0] + xlo[...]
    hhi = an[1] + xhi[...]
    he = ae[0] + ae[1]
    h1 = jnp.dot(hlo, w1lo[...], preferred_element_type=jnp.float32)
    h1 += jnp.dot(hhi, w1hi[...], preferred_element_type=jnp.float32)
    h1 += jnp.dot(he, w1e[...], preferred_element_type=jnp.float32)
    h1 = jnp.maximum(h1 + b1[...], 0.0)
    o_ref[...] = jnp.dot(h1, w2[...], preferred_element_type=jnp.float32) + b2[...]


def kernel(x, edge_index, edge_attr, W1, b1, W2, b2):
    src = edge_index[0].astype(jnp.int32).reshape(TOT_CHUNKS, CHUNK)
    dst = edge_index[1].astype(jnp.int32).reshape(TOT_CHUNKS, CHUNK)
    pad = ((0, IDX_ROWS - TOT_CHUNKS), (0, 0))
    src = jnp.pad(src, pad)
    dst = jnp.pad(dst, pad)

    rx_lo, rx_hi = pl.pallas_call(
        _prep_body,
        out_shape=(jax.ShapeDtypeStruct((N_NODES, DH), jnp.float32),
                   jax.ShapeDtypeStruct((N_NODES, DH), jnp.float32)),
    )(x)

    accn, acce = _sc_agg(rx_lo, rx_hi, src, dst, edge_attr)

    out = pl.pallas_call(
        _mlp_body,
        out_shape=jax.ShapeDtypeStruct((N_NODES, OUT), jnp.float32),
    )(accn, acce, x[:, :DH], x[:, DH:],
      W1[:DH], W1[DH:D_FEAT], W1[D_FEAT:], b1.reshape(1, HIDDEN),
      W2, b2.reshape(1, OUT))
    return out


# speedup vs baseline: 1.0005x; 1.0005x over previous
import functools

import jax
import jax.numpy as jnp
from jax import lax
from jax.experimental import pallas as pl
from jax.experimental.pallas import tpu as pltpu
from jax.experimental.pallas import tpu_sc as plsc

N_NODES = 10000
N_EDGES = 320000
D_FEAT = 128
D_EDGE = 16
HIDDEN = 128
OUT = 128

NC = 2
NS = 16
DH = D_FEAT // NC
CHUNK = 128
TOT_CHUNKS = N_EDGES // CHUNK
BASE_CH = TOT_CHUNKS // NS
EXTRA = TOT_CHUNKS - BASE_CH * NS
T_MAX = BASE_CH + 1
T_LOOP = 158
UNROLL = 2
IDX_ROWS = 2504
EDGE_SPLIT = TOT_CHUNKS // NC
NZ_S = 10
ZROWS = 125


def _sc_agg_body(rxlo_hbm, rxhi_hbm, src_hbm, dst_hbm, ea_hbm,
                 outn_hbm, oute_hbm,
                 src_v, dst_v, rows, ebuf, zbuf, zbuf16,
                 accn_sh, acce_sh, gsem, ssem, esem, tsem):
    cid = lax.axis_index("c")
    sid = lax.axis_index("s")

    zero16 = jnp.zeros((16,), jnp.float32)

    def _zrow(i, carry):
        for k in range(DH // 16):
            zbuf[i, pl.ds(k * 16, 16)] = zero16
        zbuf16[i, :] = zero16
        return carry

    rows_z = N_NODES // NZ_S

    @pl.when(sid < NZ_S)
    def _zero_fill():
        lax.fori_loop(0, ZROWS, _zrow, 0)
        for j in range(rows_z // ZROWS):
            r0 = sid * rows_z + j * ZROWS
            pltpu.sync_copy(zbuf, accn_sh.at[pl.ds(r0, ZROWS)])
            pltpu.sync_copy(zbuf16, acce_sh.at[pl.ds(r0, ZROWS)])

    plsc.subcore_barrier()

    accn_nodes = accn_sh
    acce_nodes = acce_sh

    start = sid * BASE_CH + jnp.minimum(sid, EXTRA)
    n_t = BASE_CH + (sid < EXTRA).astype(jnp.int32)

    pltpu.sync_copy(src_hbm.at[pl.ds(start, T_MAX)], src_v)
    pltpu.sync_copy(dst_hbm.at[pl.ds(start, T_MAX)], dst_v)

    def _issue_gather(t, q):
        @pl.when(cid == 0)
        def _g_lo():
            pltpu.async_copy(rxlo_hbm.at[src_v.at[t]], rows.at[q], gsem[q])

        @pl.when(cid == 1)
        def _g_hi():
            pltpu.async_copy(rxhi_hbm.at[src_v.at[t]], rows.at[q], gsem[q])

    def _edge_active(t):
        return ((start + t) // EDGE_SPLIT) == cid

    def _issue_edge(t, q):
        @pl.when(_edge_active(t))
        def _e():
            pltpu.async_copy(ea_hbm.at[pl.ds((start + t) * CHUNK, CHUNK)],
                             ebuf.at[q], esem[q])

    _issue_gather(0, 0)
    _issue_edge(0, 0)

    def _step(s, carry):
        for q in range(UNROLL):
            t = UNROLL * s + q
            q1 = (q + 1) % UNROLL

            @pl.when((t >= 1) & (t - 1 < n_t))
            def _drain():
                pltpu.make_async_copy(
                    rows.at[q1], accn_nodes.at[dst_v.at[t - 1]], ssem[q1]
                ).wait()

                @pl.when(_edge_active(t - 1))
                def _drain_e():
                    pltpu.make_async_copy(
                        ebuf.at[q1], acce_nodes.at[dst_v.at[t - 1]], tsem[q1]
                    ).wait()

            @pl.when(t + 1 < n_t)
            def _issue_next():
                _issue_gather(t + 1, q1)
                _issue_edge(t + 1, q1)

            @pl.when(t < n_t)
            def _node_part():
                pltpu.make_async_copy(rxlo_hbm.at[src_v.at[t]], rows.at[q],
                                      gsem[q]).wait()
                pltpu.async_copy(rows.at[q], accn_nodes.at[dst_v.at[t]],
                                 ssem[q], add=True)

            @pl.when((t < n_t) & _edge_active(t))
            def _edge_part():
                pltpu.make_async_copy(
                    ea_hbm.at[pl.ds((start + t) * CHUNK, CHUNK)],
                    ebuf.at[q], esem[q]).wait()

                def _relu_row(r, carry):
                    for k in range(8):
                        i = r * 8 + k
                        ebuf[q, i, :] = jnp.maximum(ebuf[q, i, :], 0.0)
                    return carry

                lax.fori_loop(0, CHUNK // 8, _relu_row, 0)
                pltpu.async_copy(ebuf.at[q], acce_nodes.at[dst_v.at[t]],
                                 tsem[q], add=True)

        return carry

    lax.fori_loop(0, T_LOOP // UNROLL, _step, 0)
    plsc.subcore_barrier()

    @pl.when(sid < NZ_S)
    def _copy_out():
        r0 = sid * rows_z
        pltpu.sync_copy(accn_sh.at[pl.ds(r0, rows_z)],
                        outn_hbm.at[cid, pl.ds(r0, rows_z)])
        pltpu.sync_copy(acce_sh.at[pl.ds(r0, rows_z)],
                        oute_hbm.at[cid, pl.ds(r0, rows_z)])


_sc_agg = functools.partial(
    pl.kernel,
    out_type=(jax.ShapeDtypeStruct((NC, N_NODES, DH), jnp.float32),
              jax.ShapeDtypeStruct((NC, N_NODES, D_EDGE), jnp.float32)),
    mesh=plsc.VectorSubcoreMesh(core_axis_name="c", subcore_axis_name="s"),
    compiler_params=pltpu.CompilerParams(use_tc_tiling_on_sc=False),
    scratch_types=[
        pltpu.VMEM((T_MAX, CHUNK), jnp.int32),
        pltpu.VMEM((T_MAX, CHUNK), jnp.int32),
        pltpu.VMEM((UNROLL, CHUNK, DH), jnp.float32),
        pltpu.VMEM((UNROLL, CHUNK, D_EDGE), jnp.float32),
        pltpu.VMEM((ZROWS, DH), jnp.float32),
        pltpu.VMEM((ZROWS, D_EDGE), jnp.float32),
        pltpu.VMEM_SHARED((N_NODES, DH), jnp.float32),
        pltpu.VMEM_SHARED((N_NODES, D_EDGE), jnp.float32),
        [pltpu.SemaphoreType.DMA] * UNROLL,
        [pltpu.SemaphoreType.DMA] * UNROLL,
        [pltpu.SemaphoreType.DMA] * UNROLL,
        [pltpu.SemaphoreType.DMA] * UNROLL,
    ],
)(_sc_agg_body)


def _prep_body(x_ref, rxlo_ref, rxhi_ref):
    rx = jnp.maximum(x_ref[...], 0.0)
    rxlo_ref[...] = rx[:, :DH]
    rxhi_ref[...] = rx[:, DH:]


def _mlp_body(an, ae, xlo, xhi, w1lo, w1hi, w1e, b1, w2, b2, o_ref):
    hlo = an[0] + xlo[...]
    hhi = an[1] + xhi[...]
    he = ae[0] + ae[1]
    h1 = jnp.dot(hlo, w1lo[...], preferred_element_type=jnp.float32)
    h1 += jnp.dot(hhi, w1hi[...], preferred_element_type=jnp.float32)
    h1 += jnp.dot(he, w1e[...], preferred_element_type=jnp.float32)
    h1 = jnp.maximum(h1 + b1[...], 0.0)
    o_ref[...] = jnp.dot(h1, w2[...], preferred_element_type=jnp.float32) + b2[...]


def kernel(x, edge_index, edge_attr, W1, b1, W2, b2):
    src = edge_index[0].astype(jnp.int32).reshape(TOT_CHUNKS, CHUNK)
    dst = edge_index[1].astype(jnp.int32).reshape(TOT_CHUNKS, CHUNK)
    pad = ((0, IDX_ROWS - TOT_CHUNKS), (0, 0))
    src = jnp.pad(src, pad)
    dst = jnp.pad(dst, pad)

    rx_lo, rx_hi = pl.pallas_call(
        _prep_body,
        out_shape=(jax.ShapeDtypeStruct((N_NODES, DH), jnp.float32),
                   jax.ShapeDtypeStruct((N_NODES, DH), jnp.float32)),
    )(x)

    accn, acce = _sc_agg(rx_lo, rx_hi, src, dst, edge_attr)

    out = pl.pallas_call(
        _mlp_body,
        out_shape=jax.ShapeDtypeStruct((N_NODES, OUT), jnp.float32),
    )(accn, acce, x[:, :DH], x[:, DH:],
      W1[:DH], W1[DH:D_FEAT], W1[D_FEAT:], b1.reshape(1, HIDDEN),
      W2, b2.reshape(1, OUT))
    return out
